# Initial kernel scaffold; baseline (speedup 1.0000x reference)
import functools

import jax
import jax.numpy as jnp
from jax import lax
from jax.experimental import pallas as pl
from jax.experimental.pallas import tpu as pltpu
from jax.experimental.pallas import tpu_sc as plsc

M = 1_000_000
B = 262_144
NC = 2
NS = 16
NW = NC * NS
B_PER_W = B // NW
IDX_ROWS_PER_W = B_PER_W // 128
ZCHUNK = 4096
NZCHUNK = (M + ZCHUNK - 1) // ZCHUNK


def _mesh():
    return plsc.VectorSubcoreMesh(core_axis_name="c", subcore_axis_name="s")


def _wid():
    return lax.axis_index("s") * NC + lax.axis_index("c")


def _make_zero_kernel():
    @functools.partial(
        pl.kernel,
        mesh=_mesh(),
        out_type=jax.ShapeDtypeStruct((M, 3), jnp.float32),
        scratch_types=[pltpu.VMEM((ZCHUNK, 3), jnp.float32)],
    )
    def zero_kernel(out_hbm, zbuf):
        wid = _wid()

        def fill(i, _):
            z16 = jnp.zeros((16,), jnp.float32)
            zbuf[pl.ds(i * 16, 16), 0] = z16
            zbuf[pl.ds(i * 16, 16), 1] = z16
            zbuf[pl.ds(i * 16, 16), 2] = z16
            return ()

        lax.fori_loop(0, ZCHUNK // 16, fill, ())

        nj = (NZCHUNK - wid + NW - 1) // NW

        def zloop(i, _):
            chunk = wid + i * NW
            r0 = jnp.where(chunk == NZCHUNK - 1, M - ZCHUNK, chunk * ZCHUNK)
            pltpu.sync_copy(zbuf, out_hbm.at[pl.ds(r0, ZCHUNK)])
            return ()

        lax.fori_loop(0, nj, zloop, ())

    return zero_kernel


def _make_scatter_kernel():
    @functools.partial(
        pl.kernel,
        mesh=_mesh(),
        out_type=(),
        scratch_types=[
            pltpu.VMEM((IDX_ROWS_PER_W, 128), jnp.int32),
            pltpu.VMEM((128, 3), jnp.float32),
            pltpu.SemaphoreType.DMA,
        ],
    )
    def scatter_kernel(buf_hbm, idx_hbm, idx_v, ones_v, sem):
        wid = _wid()

        def fill(i, _):
            o16 = jnp.ones((16,), jnp.float32)
            ones_v[pl.ds(i * 16, 16), 0] = o16
            ones_v[pl.ds(i * 16, 16), 1] = o16
            ones_v[pl.ds(i * 16, 16), 2] = o16
            return ()

        lax.fori_loop(0, 8, fill, ())

        pltpu.sync_copy(
            idx_hbm.at[pl.ds(wid * IDX_ROWS_PER_W, IDX_ROWS_PER_W)], idx_v
        )
        copies = [
            pltpu.async_copy(ones_v, buf_hbm.at[idx_v.at[j]], sem)
            for j in range(IDX_ROWS_PER_W)
        ]
        for c in copies:
            c.wait()

    return scatter_kernel


def kernel(vertices, mask):
    del vertices
    idx = mask.astype(jnp.int32).reshape(NW * IDX_ROWS_PER_W, 128)
    zeros = _make_zero_kernel()()
    buf = jax.new_ref(zeros)
    _make_scatter_kernel()(buf, idx)
    return buf[...].reshape(1, M, 3)


# speedup vs baseline: 1.5895x; 1.5895x over previous
import functools

import jax
import jax.numpy as jnp
from jax import lax
from jax.experimental import pallas as pl
from jax.experimental.pallas import tpu as pltpu
from jax.experimental.pallas import tpu_sc as plsc
from jax._src.pallas import mpmd as _mpmd

M = 1_000_000
B = 262_144
NC = 2
NS = 16
NW = NC * NS
E = 3 * M
BE = 3 * B
IDXC = 128
NCHUNK = BE // IDXC
C_PER_W = NCHUNK // NW
ZCHUNK = 12288
NZCHUNK = (E + ZCHUNK - 1) // ZCHUNK


def _mesh():
    return plsc.VectorSubcoreMesh(core_axis_name="c", subcore_axis_name="s")


_PARAMS = pltpu.CompilerParams(use_tc_tiling_on_sc=False)


def _wid():
    return lax.axis_index("s") * NC + lax.axis_index("c")


def _make_zero_kernel():
    @functools.partial(
        pl.kernel,
        mesh=_mesh(),
        out_type=jax.ShapeDtypeStruct((E,), jnp.float32),
        scratch_types=[pltpu.VMEM((ZCHUNK,), jnp.float32)],
        compiler_params=_PARAMS,
    )
    def zero_kernel(zc_hbm, out_hbm, zbuf):
        wid = _wid()
        pltpu.sync_copy(zc_hbm, zbuf)
        nj = (NZCHUNK - wid + NW - 1) // NW

        def zloop(i, _):
            chunk = wid + i * NW
            e0 = jnp.where(chunk == NZCHUNK - 1, E - ZCHUNK, chunk * ZCHUNK)
            pltpu.sync_copy(zbuf, out_hbm.at[pl.ds(e0, ZCHUNK)])
            return ()

        lax.fori_loop(0, nj, zloop, ())

    return zero_kernel


def _make_scatter_kernel():
    def scatter_body(buf_in, idx_hbm, ones_hbm, out_hbm, idx_v, ones_v, sem):
        del buf_in
        wid = _wid()
        pltpu.sync_copy(ones_hbm, ones_v)

        def sloop(j, _):
            pltpu.sync_copy(idx_hbm.at[wid * C_PER_W + j], idx_v)
            pltpu.async_copy(ones_v, out_hbm.at[idx_v], sem).wait()
            return ()

        lax.fori_loop(0, C_PER_W, sloop, ())

    return _mpmd._mpmd_map(
        [(_mesh(), scatter_body)],
        out_types=jax.ShapeDtypeStruct((E,), jnp.float32),
        input_output_aliases={0: 0},
        scratch_types=[
            pltpu.VMEM((IDXC,), jnp.int32),
            pltpu.VMEM((IDXC,), jnp.float32),
            pltpu.SemaphoreType.DMA,
        ],
        compiler_params=_PARAMS,
    )


def kernel(vertices, mask):
    del vertices
    idx = mask.astype(jnp.int32)
    idx3 = (3 * idx[:, None] + jnp.arange(3, dtype=jnp.int32)[None, :])
    idx3 = idx3.reshape(NCHUNK, IDXC)
    zconst = jnp.zeros((ZCHUNK,), jnp.float32)
    ones = jnp.ones((IDXC,), jnp.float32)
    zeros = _make_zero_kernel()(zconst)
    out = _make_scatter_kernel()(zeros, idx3, ones)
    return out.reshape(1, M, 3)
